# baseline (speedup 1.0000x reference)
import jax
import jax.numpy as jnp
from jax.experimental import pallas as pl

_SEQ_BLK = 128


def _add_kernel(x_ref, pos_ref, out_ref):
    out_ref[...] = x_ref[...] + pos_ref[...][None, :, :]


def kernel(x, pos_table):
    batch, seq_len, embed = x.shape
    grid = (seq_len // _SEQ_BLK,)
    return pl.pallas_call(
        _add_kernel,
        grid=grid,
        in_specs=[
            pl.BlockSpec((batch, _SEQ_BLK, embed), lambda s: (0, s, 0)),
            pl.BlockSpec((_SEQ_BLK, embed), lambda s: (s, 0)),
        ],
        out_specs=pl.BlockSpec((batch, _SEQ_BLK, embed), lambda s: (0, s, 0)),
        out_shape=jax.ShapeDtypeStruct((batch, seq_len, embed), x.dtype),
    )(x, pos_table[:seq_len])


# speedup vs baseline: 1.0832x; 1.0832x over previous
import jax
import jax.numpy as jnp
from jax.experimental import pallas as pl

_SEQ_BLK = 512


def _add_kernel(x_ref, pos_ref, out_ref):
    out_ref[...] = x_ref[...] + pos_ref[...][None, :, :]


def kernel(x, pos_table):
    batch, seq_len, embed = x.shape
    grid = (seq_len // _SEQ_BLK,)
    return pl.pallas_call(
        _add_kernel,
        grid=grid,
        in_specs=[
            pl.BlockSpec((batch, _SEQ_BLK, embed), lambda s: (0, s, 0)),
            pl.BlockSpec((_SEQ_BLK, embed), lambda s: (s, 0)),
        ],
        out_specs=pl.BlockSpec((batch, _SEQ_BLK, embed), lambda s: (0, s, 0)),
        out_shape=jax.ShapeDtypeStruct((batch, seq_len, embed), x.dtype),
    )(x, pos_table[:seq_len])


# speedup vs baseline: 1.0887x; 1.0051x over previous
import jax
import jax.numpy as jnp
from jax.experimental import pallas as pl
from jax.experimental.pallas import tpu as pltpu

_SEQ_BLK = 256


def _add_kernel(x_ref, pos_ref, out_ref):
    out_ref[...] = x_ref[...] + pos_ref[...][None, :, :]


def kernel(x, pos_table):
    batch, seq_len, embed = x.shape
    grid = (seq_len // _SEQ_BLK,)
    return pl.pallas_call(
        _add_kernel,
        grid=grid,
        in_specs=[
            pl.BlockSpec((batch, _SEQ_BLK, embed), lambda s: (0, s, 0)),
            pl.BlockSpec((_SEQ_BLK, embed), lambda s: (s, 0)),
        ],
        out_specs=pl.BlockSpec((batch, _SEQ_BLK, embed), lambda s: (0, s, 0)),
        out_shape=jax.ShapeDtypeStruct((batch, seq_len, embed), x.dtype),
        compiler_params=pltpu.CompilerParams(
            dimension_semantics=("parallel",),
        ),
    )(x, pos_table[:seq_len])
